# Initial kernel scaffold; baseline (speedup 1.0000x reference)
import jax
import jax.numpy as jnp
from jax.experimental import pallas as pl


def kernel(pred, y, mid_pos, mid_vel, vol, rho_0, h, dt, y_mean, y_std, num_boundary_particles):
    raise NotImplementedError("write your pallas kernel here")


# speedup vs baseline: 1.5560x; 1.5560x over previous
import jax
import jax.numpy as jnp
from jax.experimental import pallas as pl
from jax.experimental.pallas import tpu as pltpu

_ALPHA = 1.0
_BETA = 0.5
_GAMMA = 0.5
_EPS = 1e-12
_BI = 256


def _loss_block_kernel(scal_ref, predb_ref, yb_ref, posb_ref, velb_ref,
                       posT_ref, velT_ref, volT_ref, out_ref):
    i = pl.program_id(0)
    n_total = velT_ref.shape[1]

    rho_0 = scal_ref[0, 0]
    h = scal_ref[0, 1]
    hinv = 1.0 / h
    sigma = 8.0 / (jnp.float32(jnp.pi) * h * h * h)

    xb = posb_ref[...]
    vb = velb_ref[...]
    xT = posT_ref[...]
    vT = velT_ref[...]
    vol = volT_ref[...]

    d2 = None
    P = None
    Q = None
    for k in range(3):
        diffk = xb[:, k:k + 1] - xT[k:k + 1, :]
        t = diffk * diffk
        pk = xb[:, k:k + 1] * vT[k:k + 1, :]
        qk = vb[:, k:k + 1] * xT[k:k + 1, :]
        d2 = t if d2 is None else d2 + t
        P = pk if P is None else P + pk
        Q = qk if Q is None else Q + qk
    d = jnp.sqrt(d2 + _EPS)

    s_i = jnp.sum(xb * vb, axis=1, keepdims=True)
    s_j = jnp.sum(xT * vT, axis=0, keepdims=True)

    q = d * hinv
    near = q <= 0.5
    valid = q <= 1.0
    one_m_q = jnp.maximum(1.0 - q, 0.0)

    w_near = 6.0 * (q * q * q - q * q) + 1.0
    w_far = 2.0 * one_m_q * one_m_q * one_m_q
    W = sigma * jnp.where(valid, jnp.where(near, w_near, w_far), 0.0)

    g_near = 6.0 * (3.0 * q * q - 2.0 * q)
    g_far = -6.0 * one_m_q * one_m_q
    dW = (sigma * hinv) * jnp.where(valid, jnp.where(near, g_near, g_far), 0.0)

    rho = rho_0 * jnp.sum(vol * W, axis=1)
    cmp = rho / rho_0 - 1.0
    b2 = jnp.sum(jnp.abs(cmp))

    dot = (P + Q) - s_i - s_j
    div = rho_0 * jnp.sum(vol * (dW * dot / (d + _EPS)), axis=1)
    b3 = jnp.sum(jnp.abs(div))

    dy = yb_ref[...] - predb_ref[...]
    b1 = jnp.sum(dy * dy)

    contrib = (_ALPHA * b1 + _BETA * b2 + _GAMMA * b3) / n_total

    @pl.when(i == 0)
    def _():
        out_ref[...] = jnp.zeros((1, 1), jnp.float32)

    out_ref[...] += jnp.reshape(contrib, (1, 1))


def kernel(pred, y, mid_pos, mid_vel, vol, rho_0, h, dt, y_mean, y_std,
           num_boundary_particles):
    n = pred.shape[0]
    y_inv = y * y_std + y_mean
    free = (jnp.arange(n) >= num_boundary_particles).astype(pred.dtype)[:, None]
    pos = mid_pos + free * y_inv
    vel = mid_vel + free * (y_inv / dt)

    posT = pos.T
    velT = vel.T
    volT = vol.reshape(1, n)
    scal = jnp.stack([jnp.asarray(rho_0, jnp.float32),
                      jnp.asarray(h, jnp.float32)]).reshape(1, 2)

    grid = (n // _BI,)
    out = pl.pallas_call(
        _loss_block_kernel,
        grid=grid,
        in_specs=[
            pl.BlockSpec(memory_space=pltpu.SMEM),
            pl.BlockSpec((_BI, 3), lambda i: (i, 0)),
            pl.BlockSpec((_BI, 3), lambda i: (i, 0)),
            pl.BlockSpec((_BI, 3), lambda i: (i, 0)),
            pl.BlockSpec((_BI, 3), lambda i: (i, 0)),
            pl.BlockSpec((3, n), lambda i: (0, 0)),
            pl.BlockSpec((3, n), lambda i: (0, 0)),
            pl.BlockSpec((1, n), lambda i: (0, 0)),
        ],
        out_specs=pl.BlockSpec((1, 1), lambda i: (0, 0)),
        out_shape=jax.ShapeDtypeStruct((1, 1), jnp.float32),
    )(scal, pred, y, pos, vel, posT, velT, volT)
    return out.reshape(())


# speedup vs baseline: 1.6974x; 1.0909x over previous
import jax
import jax.numpy as jnp
from jax.experimental import pallas as pl
from jax.experimental.pallas import tpu as pltpu

_ALPHA = 1.0
_BETA = 0.5
_GAMMA = 0.5
_EPS = 1e-12
_BI = 256


def _loss_kernel(scal_ref, pred_ref, y_ref, mpos_ref, mvel_ref,
                 yT_ref, mposT_ref, mvelT_ref, vols_ref,
                 ystd_row_ref, ymean_row_ref, ystd_col_ref, ymean_col_ref,
                 out_ref, posT_s, velT_s, sj_s):
    i = pl.program_id(0)
    n_total = yT_ref.shape[1]

    rho_0 = scal_ref[0, 0]
    h = scal_ref[0, 1]
    dt = scal_ref[0, 2]
    nbp = scal_ref[0, 3]
    hinv = 1.0 / h
    dtinv = 1.0 / dt

    @pl.when(i == 0)
    def _():
        y_invT = yT_ref[...] * ystd_col_ref[...] + ymean_col_ref[...]
        lane = jax.lax.broadcasted_iota(jnp.int32, (1, n_total), 1)
        freeT = lane >= nbp.astype(jnp.int32)
        zT = jnp.zeros_like(y_invT)
        posT_s[...] = mposT_ref[...] + jnp.where(freeT, y_invT, zT)
        velT_s[...] = mvelT_ref[...] + jnp.where(freeT, y_invT * dtinv, zT)
        pT = posT_s[...]
        vT = velT_s[...]
        sj_s[...] = (pT[0:1, :] * vT[0:1, :] + pT[1:2, :] * vT[1:2, :]
                     + pT[2:3, :] * vT[2:3, :])
        out_ref[...] = jnp.zeros((1, 1), jnp.float32)

    rows = pl.ds(i * _BI, _BI)
    yb = y_ref[rows, :]
    yb_inv = yb * ystd_row_ref[...] + ymean_row_ref[...]
    riota = jax.lax.broadcasted_iota(jnp.int32, (_BI, 1), 0)
    freeb = (riota + _BI * i) >= nbp.astype(jnp.int32)
    zb = jnp.zeros_like(yb_inv)
    xb = mpos_ref[rows, :] + jnp.where(freeb, yb_inv, zb)
    vb = mvel_ref[rows, :] + jnp.where(freeb, yb_inv * dtinv, zb)

    xT = posT_s[...]
    vT = velT_s[...]
    vols = vols_ref[...]

    d2 = None
    PQ = None
    for k in range(3):
        diffk = xb[:, k:k + 1] - xT[k:k + 1, :]
        t = diffk * diffk
        c = xb[:, k:k + 1] * vT[k:k + 1, :] + vb[:, k:k + 1] * xT[k:k + 1, :]
        d2 = t if d2 is None else d2 + t
        PQ = c if PQ is None else PQ + c

    s_i = jnp.sum(xb * vb, axis=1, keepdims=True)

    d2p = d2 + _EPS
    rinv = jax.lax.rsqrt(d2p)
    d = d2p * rinv
    q = d * hinv

    q2 = q * q
    near = q <= 0.5
    u = jnp.maximum(1.0 - q, 0.0)
    u2 = u * u

    w_near = 6.0 * (q2 * (q - 1.0)) + 1.0
    w_far = (2.0 * u) * u2
    Wt = jnp.where(near, w_near, w_far)

    g_near = 18.0 * q2 - 12.0 * q
    g_far = -6.0 * u2
    Gt = jnp.where(near, g_near, g_far)

    S2 = jnp.sum(vols * Wt, axis=1)
    rho = rho_0 * S2
    cmp = rho / rho_0 - 1.0
    b2 = jnp.sum(jnp.abs(cmp))

    dot = PQ - s_i - sj_s[...]
    S3 = jnp.sum(vols * (Gt * dot * rinv), axis=1)
    b3 = jnp.sum(jnp.abs((rho_0 * hinv) * S3))

    dy = y_ref[rows, :] - pred_ref[rows, :]
    b1 = jnp.sum(dy * dy)

    contrib = (_ALPHA * b1 + _BETA * b2 + _GAMMA * b3) / n_total
    out_ref[...] += jnp.reshape(contrib, (1, 1))


def kernel(pred, y, mid_pos, mid_vel, vol, rho_0, h, dt, y_mean, y_std,
           num_boundary_particles):
    n = pred.shape[0]
    f32 = jnp.float32
    sigma = 8.0 / (f32(jnp.pi) * h * h * h)
    vols = (vol * sigma).reshape(1, n)
    scal = jnp.stack([jnp.asarray(rho_0, f32), jnp.asarray(h, f32),
                      jnp.asarray(dt, f32),
                      jnp.asarray(num_boundary_particles, f32)]).reshape(1, 4)

    full_n3 = pl.BlockSpec((n, 3), lambda i: (0, 0))
    full_3n = pl.BlockSpec((3, n), lambda i: (0, 0))

    out = pl.pallas_call(
        _loss_kernel,
        grid=(n // _BI,),
        in_specs=[
            pl.BlockSpec(memory_space=pltpu.SMEM),
            full_n3, full_n3, full_n3, full_n3,
            full_3n, full_3n, full_3n,
            pl.BlockSpec((1, n), lambda i: (0, 0)),
            pl.BlockSpec((1, 3), lambda i: (0, 0)),
            pl.BlockSpec((1, 3), lambda i: (0, 0)),
            pl.BlockSpec((3, 1), lambda i: (0, 0)),
            pl.BlockSpec((3, 1), lambda i: (0, 0)),
        ],
        out_specs=pl.BlockSpec((1, 1), lambda i: (0, 0)),
        out_shape=jax.ShapeDtypeStruct((1, 1), jnp.float32),
        scratch_shapes=[
            pltpu.VMEM((3, n), jnp.float32),
            pltpu.VMEM((3, n), jnp.float32),
            pltpu.VMEM((1, n), jnp.float32),
        ],
    )(scal, pred, y, mid_pos, mid_vel,
      y.T, mid_pos.T, mid_vel.T, vols,
      y_std.reshape(1, 3), y_mean.reshape(1, 3),
      y_std.reshape(3, 1), y_mean.reshape(3, 1))
    return out.reshape(())
